# baseline (speedup 1.0000x reference)
import jax
import jax.numpy as jnp
from jax import lax
from jax.experimental import pallas as pl
from jax.experimental.pallas import tpu as pltpu
from jax.experimental.pallas import tpu_sc as plsc

_H = 16
_L = 2048
_TW = 4096
_SLOT = 4096
_NSL = 8
_HPC = 8
_BLK = _L // 128


def _sc_body(tab_hbm, out_hbm, col_v, tab_v, sem):
    c = lax.axis_index("c")
    t = lax.axis_index("s")
    lanes = lax.iota(jnp.int32, 16)

    qoff, ia, ib, msk = [], [], [], []
    for u in range(_NSL):
        sh = 8 * t + 1 + u
        r = sh % 16
        qoff.append((sh // 16) * 16)
        ia.append((16 - r + lanes) % 16)
        ib.append((lanes - r) % 16)
        msk.append(lanes < r)

    def bchunk_group(p, k_lo, k_hi):
        base_p = p * (_NSL * _SLOT)

        def bchunk(k, cc):
            x = 128 + k * 16
            for u in range(_NSL):
                off_b = pl.multiple_of(x - qoff[u], 16)
                off_a = pl.multiple_of(jnp.maximum(x - qoff[u] - 16, 0), 16)
                va = col_v[pl.ds(off_a, 16)]
                vb = col_v[pl.ds(off_b, 16)]
                pa = jnp.take(va, ia[u], mode="wrap")
                pb = jnp.take(vb, ib[u], mode="wrap")
                w = jnp.where(msk[u], pa, pb)
                tab_v[pl.ds(pl.multiple_of(base_p + u * _SLOT + x, 16), 16)] = w
            return cc

        lax.fori_loop(k_lo, k_hi, bchunk, 0)

    _NCH = (_TW - 128) // 16
    _GRP = 8
    _GRP2 = -(-(_NCH - _BLK * _GRP) // _BLK)

    pltpu.sync_copy(tab_hbm.at[c * _HPC], col_v)
    bchunk_group(0, 0, _NCH)

    def head_iter(hh, carry):
        p = hh % 2
        h = c * _HPC + hh

        @pl.when(hh + 1 < _HPC)
        def _():
            pltpu.sync_copy(tab_hbm.at[c * _HPC + hh + 1], col_v)

        cps = []
        for b in range(_BLK):
            for u in range(_NSL):
                i = 128 * b + 8 * t + u
                src = pl.multiple_of(
                    p * (_NSL * _SLOT) + u * _SLOT + (_L - 128 * b), 128
                )
                cps.append(
                    pltpu.async_copy(
                        tab_v.at[pl.ds(src, _L)], out_hbm.at[h, i], sem
                    )
                )
            k_lo, k_hi = b * _GRP, (b + 1) * _GRP

            @pl.when(hh + 1 < _HPC)
            def _(k_lo=k_lo, k_hi=k_hi):
                bchunk_group(1 - p, k_lo, k_hi)

        base2 = _BLK * _GRP
        for b in range(_BLK):
            for cp in cps[b * _NSL : (b + 1) * _NSL]:
                cp.wait()
            k_lo = base2 + b * _GRP2
            k_hi = min(base2 + (b + 1) * _GRP2, _NCH)
            if k_lo < k_hi:

                @pl.when(hh + 1 < _HPC)
                def _(k_lo=k_lo, k_hi=k_hi):
                    bchunk_group(1 - p, k_lo, k_hi)
        return carry

    lax.fori_loop(0, _HPC, head_iter, 0)


def kernel(L, relative_bias):
    del L
    tab_t = jnp.zeros((_H, _TW), jnp.float32).at[:, : 2 * _L - 1].set(relative_bias.T)
    mesh = plsc.VectorSubcoreMesh(core_axis_name="c", subcore_axis_name="s")
    run = pl.kernel(
        _sc_body,
        out_type=jax.ShapeDtypeStruct((_H, _L, _L), jnp.float32),
        mesh=mesh,
        scratch_types=[
            pltpu.VMEM((_TW,), jnp.float32),
            pltpu.VMEM((2 * _NSL * _SLOT,), jnp.float32),
            pltpu.SemaphoreType.DMA,
        ],
    )
    return run(tab_t)


# speedup vs baseline: 1.1367x; 1.1367x over previous
import jax
import jax.numpy as jnp
from jax import lax
from jax.experimental import pallas as pl
from jax.experimental.pallas import tpu as pltpu
from jax.experimental.pallas import tpu_sc as plsc

_H = 16
_L = 2048
_TW = 4096
_SLOT = 4096
_NSL = 8
_HPC = 8
_BLK = _L // 128


def _sc_body(tab_hbm, out_hbm, col_v, tab_v, sem):
    c = lax.axis_index("c")
    t = lax.axis_index("s")
    lanes = lax.iota(jnp.int32, 16)

    qoff, ia, ib, msk = [], [], [], []
    for u in range(_NSL):
        sh = 8 * t + 1 + u
        r = sh % 16
        qoff.append((sh // 16) * 16)
        ia.append((16 - r + lanes) % 16)
        ib.append((lanes - r) % 16)
        msk.append(lanes < r)

    def bchunk_group(p, k_lo, k_hi):
        def bchunk(k, cc):
            x = 128 + k * 16
            for u in range(_NSL):
                off_b = pl.multiple_of(x - qoff[u], 16)
                off_a = pl.multiple_of(jnp.maximum(x - qoff[u] - 16, 0), 16)
                va = col_v[pl.ds(off_a, 16)]
                vb = col_v[pl.ds(off_b, 16)]
                pa = jnp.take(va, ia[u], mode="wrap")
                pb = jnp.take(vb, ib[u], mode="wrap")
                w = jnp.where(msk[u], pa, pb)
                tab_v[p, u, pl.ds(pl.multiple_of(x, 16), 16)] = w
            return cc

        lax.fori_loop(k_lo, k_hi, bchunk, 0)

    _NCH = (_TW - 128) // 16
    _GRP = -(-_NCH // _BLK)

    pltpu.sync_copy(tab_hbm.at[c * _HPC], col_v)
    bchunk_group(0, 0, _NCH)

    def head_iter(hh, carry):
        p = hh % 2
        h = c * _HPC + hh

        @pl.when(hh + 1 < _HPC)
        def _():
            pltpu.sync_copy(tab_hbm.at[c * _HPC + hh + 1], col_v)

        cps = []
        for b in range(_BLK):
            src_c = pl.multiple_of(_L - 128 * b, 128)
            dst_r = pl.multiple_of(128 * b + 8 * t, 8)
            cps.append(
                pltpu.async_copy(
                    tab_v.at[p, pl.ds(0, _NSL), pl.ds(src_c, _L)],
                    out_hbm.at[h, pl.ds(dst_r, _NSL), :],
                    sem,
                )
            )
            k_lo, k_hi = b * _GRP, min((b + 1) * _GRP, _NCH)
            if k_lo < k_hi:

                @pl.when(hh + 1 < _HPC)
                def _(k_lo=k_lo, k_hi=k_hi):
                    bchunk_group(1 - p, k_lo, k_hi)

        for cp in cps:
            cp.wait()
        return carry

    lax.fori_loop(0, _HPC, head_iter, 0)


def kernel(L, relative_bias):
    del L
    tab_t = jnp.zeros((_H, _TW), jnp.float32).at[:, : 2 * _L - 1].set(relative_bias.T)
    mesh = plsc.VectorSubcoreMesh(core_axis_name="c", subcore_axis_name="s")
    run = pl.kernel(
        _sc_body,
        out_type=jax.ShapeDtypeStruct((_H, _L, _L), jnp.float32),
        mesh=mesh,
        scratch_types=[
            pltpu.VMEM((_TW,), jnp.float32),
            pltpu.VMEM((2, _NSL, _SLOT), jnp.float32),
            pltpu.SemaphoreType.DMA,
        ],
    )
    return run(tab_t)
